# baseline (speedup 1.0000x reference)
import dataclasses
import functools

import jax
import jax.numpy as jnp
from jax import lax
from jax.experimental import pallas as pl
from jax.experimental.pallas import tpu as pltpu
from jax.experimental.pallas import tpu_sc as plsc

N = 10000
E = 320000
D = 128
NC = 2
NS = 16
NW = NC * NS
L = 16
C = 128
EPT = E // NW
EPT_PAD = 10240
NPADE = EPT_PAD - EPT
NCHUNK = EPT_PAD // C
NPAD = 10240
RPT = NPAD // NS
HALF = 5120
DUMP = 512
ACCR = HALF + DUMP
ZR = ACCR // NS // 4
WPT = HALF // NS

_MESH = dict(core_axis_name="c", subcore_axis_name="s")


def _sc_compiler_params():
    cp = pltpu.CompilerParams()
    if "needs_layout_passes" in pltpu.CompilerParams.__dataclass_fields__:
        cp = dataclasses.replace(cp, needs_layout_passes=False)
    return cp


def _proj_tc(x_pad, W, a_l, a_r):
    def body(x_ref, w_ref, al_ref, ar_ref, feat_ref, el_ref, er_ref):
        feat = jnp.dot(x_ref[...], w_ref[...], preferred_element_type=jnp.float32)
        feat_ref[...] = feat
        el_ref[...] = jnp.dot(feat, al_ref[...], preferred_element_type=jnp.float32)
        er_ref[...] = jnp.dot(feat, ar_ref[...], preferred_element_type=jnp.float32)

    feat, el, er = pl.pallas_call(
        body,
        out_shape=(
            jax.ShapeDtypeStruct((NPAD, D), jnp.float32),
            jax.ShapeDtypeStruct((NPAD, 1), jnp.float32),
            jax.ShapeDtypeStruct((NPAD, 1), jnp.float32),
        ),
    )(x_pad, W, a_l.reshape(D, 1), a_r.reshape(D, 1))
    return feat, el[:, 0], er[:, 0]


def _combine_elu_tc(p):
    def body(p_ref, h_ref):
        for hh in range(2):
            z = p_ref[0, hh] + p_ref[1, hh]
            h_ref[hh * HALF:(hh + 1) * HALF, :] = jnp.where(
                z > 0, z, jnp.exp(jnp.minimum(z, 0.0)) - 1.0)

    return pl.pallas_call(
        body,
        out_shape=jax.ShapeDtypeStruct((NPAD, D), jnp.float32),
    )(p)


def _combine_final_tc(p):
    def body(p_ref, o_ref):
        o_ref[:HALF, :] = p_ref[0, 0] + p_ref[1, 0]
        o_ref[HALF:, :] = (p_ref[0, 1, :N - HALF, :] + p_ref[1, 1, :N - HALF, :])

    return pl.pallas_call(
        body,
        out_shape=jax.ShapeDtypeStruct((N, D), jnp.float32),
    )(p)


def _sc_scores(el, er, src_r, dst_r):
    mesh = plsc.VectorSubcoreMesh(**_MESH)

    @functools.partial(
        pl.kernel,
        out_type=(
            jax.ShapeDtypeStruct((NW, NCHUNK, C), jnp.float32),
            jax.ShapeDtypeStruct((NC, NPAD), jnp.float32),
        ),
        mesh=mesh,
        compiler_params=_sc_compiler_params(),
        scratch_types=[
            pltpu.VMEM((NPAD,), jnp.float32),
            pltpu.VMEM((NPAD,), jnp.float32),
            pltpu.VMEM((NCHUNK, C), jnp.int32),
            pltpu.VMEM((NCHUNK, C), jnp.int32),
            pltpu.VMEM((NCHUNK, C), jnp.float32),
            pltpu.VMEM((RPT,), jnp.float32),
            pltpu.VMEM_SHARED((NPAD,), jnp.float32),
        ],
    )
    def k(el_hbm, er_hbm, src_hbm, dst_hbm, w_hbm, s_hbm,
          el_v, er_v, src_v, dst_v, w_v, z_v, s_sh):
        cid = lax.axis_index("c")
        sid = lax.axis_index("s")
        wid = cid * NS + sid

        zero16 = jnp.zeros((L,), jnp.float32)

        @pl.loop(0, RPT // L)
        def _(i):
            z_v[pl.ds(i * L, L)] = zero16

        pltpu.sync_copy(z_v, s_sh.at[pl.ds(sid * RPT, RPT)])

        pltpu.sync_copy(el_hbm, el_v)
        pltpu.sync_copy(er_hbm, er_v)
        pltpu.sync_copy(src_hbm.at[wid], src_v)
        pltpu.sync_copy(dst_hbm.at[wid], dst_v)

        plsc.subcore_barrier()

        @pl.loop(0, NCHUNK)
        def _(j):
            for kk in range(C // L):
                sl = pl.ds(kk * L, L)
                s16 = src_v[j, sl]
                d16 = dst_v[j, sl]
                z = plsc.load_gather(el_v, [s16]) + plsc.load_gather(er_v, [d16])
                e = jnp.where(z >= 0, z, 0.2 * z)
                w_v[j, sl] = jnp.exp(e)
            pltpu.sync_copy(w_v.at[j], s_sh.at[dst_v.at[j]], add=True)

        pltpu.sync_copy(w_v, w_hbm.at[wid])

        plsc.subcore_barrier()

        pltpu.sync_copy(s_sh.at[pl.ds(sid * RPT, RPT)],
                        s_hbm.at[cid, pl.ds(sid * RPT, RPT)])

    return k(el, er, src_r, dst_r)


def _sc_aggregate(featp, w, s, src_r, dst_r):
    mesh = plsc.VectorSubcoreMesh(**_MESH)

    @functools.partial(
        pl.kernel,
        out_type=jax.ShapeDtypeStruct((NC, 2, HALF, D), jnp.float32),
        mesh=mesh,
        compiler_params=_sc_compiler_params(),
        scratch_types=[
            pltpu.VMEM((NPAD,), jnp.float32),
            pltpu.VMEM((NPAD,), jnp.float32),
            pltpu.VMEM((NCHUNK, C), jnp.int32),
            pltpu.VMEM((NCHUNK, C), jnp.int32),
            pltpu.VMEM((NCHUNK, C), jnp.float32),
            pltpu.VMEM((C,), jnp.int32),
            pltpu.VMEM((C,), jnp.int32),
            pltpu.VMEM((C, D), jnp.float32),
            pltpu.VMEM((C, D), jnp.float32),
            pltpu.SemaphoreType.DMA,
            pltpu.SemaphoreType.DMA,
            pltpu.SemaphoreType.DMA,
            pltpu.SemaphoreType.DMA,
            pltpu.VMEM_SHARED((ACCR, D), jnp.float32),
        ],
    )
    def k(feat_hbm, w_hbm, s_hbm, src_hbm, dst_hbm, out_hbm,
          sinv_v, tmp_v, src_v, dst_v, al_v,
          idx0, idx1, rows0, rows1,
          semg0, semg1, sems0, sems1, acc_sh):
        idx = (idx0, idx1)
        rows = (rows0, rows1)
        semg = (semg0, semg1)
        sems = (sems0, sems1)
        cid = lax.axis_index("c")
        sid = lax.axis_index("s")
        wid = cid * NS + sid

        zero16 = jnp.zeros((L,), jnp.float32)

        def zero_rows0():
            @pl.loop(0, C)
            def _(r):
                for cc in range(D // L):
                    rows[0][r, pl.ds(cc * L, L)] = zero16

        def zero_acc():
            base = sid * (ACCR // NS)
            for off, nrow in ((0, C), (C, C), (2 * C, ACCR // NS - 2 * C)):
                pltpu.sync_copy(rows[0].at[pl.ds(0, nrow)],
                                acc_sh.at[pl.ds(base + off, nrow)])

        zero_rows0()
        zero_acc()

        pltpu.sync_copy(s_hbm.at[0], sinv_v)
        pltpu.sync_copy(s_hbm.at[1], tmp_v)

        @pl.loop(0, NPAD // L)
        def _(i):
            sl = pl.ds(i * L, L)
            sinv_v[sl] = 1.0 / (sinv_v[sl] + tmp_v[sl] + 1e-9)

        pltpu.sync_copy(src_hbm.at[wid], src_v)
        pltpu.sync_copy(dst_hbm.at[wid], dst_v)
        pltpu.sync_copy(w_hbm.at[wid], al_v)

        @pl.loop(0, NCHUNK)
        def _(j):
            for kk in range(C // L):
                sl = pl.ds(kk * L, L)
                g = plsc.load_gather(sinv_v, [dst_v[j, sl]])
                al_v[j, sl] = al_v[j, sl] * g

        plsc.subcore_barrier()

        iota16 = lax.iota(jnp.int32, L)

        def gather_issue(j, p):
            pltpu.async_copy(feat_hbm.at[src_v.at[j]], rows[p], semg[p])

        def gather_wait(j, p):
            pltpu.make_async_copy(feat_hbm.at[src_v.at[j]], rows[p],
                                  semg[p]).wait()

        def scatter_issue(j, p):
            pltpu.async_copy(rows[p], acc_sh.at[idx[p]], sems[p], add=True)

        def scatter_drain(p):
            pltpu.make_async_copy(rows[p], acc_sh.at[idx[p]], sems[p]).wait()

        for h in range(2):
            gather_issue(0, 0)

            @pl.loop(0, NCHUNK // 2)
            def _(t):
                base = t * 2
                for p in range(2):
                    j = base + p
                    gather_wait(j, p)

                    pn = (p + 1) % 2

                    @pl.when(j + 1 < NCHUNK)
                    def _():
                        gather_issue(j + 1, pn)

                    for kk in range(C // L):
                        sl = pl.ds(kk * L, L)
                        local = dst_v[j, sl] - (h * HALF)
                        dump = HALF + ((j * C + kk * L + iota16) & (DUMP - 1))
                        ok = (local >= 0) & (local < HALF)
                        idx[p][sl] = jnp.where(ok, local, dump)

                    @pl.loop(0, C)
                    def _(r):
                        a16 = plsc.load_gather(al_v.at[j],
                                               [jnp.full((L,), r, jnp.int32)])
                        for cc in range(D // L):
                            sl = pl.ds(cc * L, L)
                            rows[p][r, sl] = rows[p][r, sl] * a16

                    pltpu.sync_copy(rows[p], acc_sh.at[idx[p]], add=True)

            plsc.subcore_barrier()

            pltpu.sync_copy(acc_sh.at[pl.ds(sid * WPT, WPT)],
                            out_hbm.at[cid, h, pl.ds(sid * WPT, WPT)])

            if h == 0:
                zero_rows0()
                zero_acc()
                plsc.subcore_barrier()

    return k(featp, w, s, src_r, dst_r)


def _gat_layer(x_pad, src_r, dst_r, W, a_l, a_r):
    featp, el, er = _proj_tc(x_pad, W, a_l, a_r)
    w, s = _sc_scores(el, er, src_r, dst_r)
    return _sc_aggregate(featp, w, s, src_r, dst_r)


def kernel(x, edge_index, W0, a_l0, a_r0, W1, a_l1, a_r1):
    x_pad = jnp.pad(x, ((0, NPAD - N), (0, 0)))
    src_r = jnp.concatenate(
        [edge_index[0].reshape(NW, EPT),
         jnp.zeros((NW, NPADE), jnp.int32)], axis=1).reshape(NW, NCHUNK, C)
    dst_r = jnp.concatenate(
        [edge_index[1].reshape(NW, EPT),
         jnp.full((NW, NPADE), N, jnp.int32)], axis=1).reshape(NW, NCHUNK, C)
    p0 = _gat_layer(x_pad, src_r, dst_r, W0, a_l0, a_r0)
    h = _combine_elu_tc(p0)
    p1 = _gat_layer(h, src_r, dst_r, W1, a_l1, a_r1)
    return _combine_final_tc(p1)


# speedup vs baseline: 1.0909x; 1.0909x over previous
import dataclasses
import functools

import jax
import jax.numpy as jnp
from jax import lax
from jax.experimental import pallas as pl
from jax.experimental.pallas import tpu as pltpu
from jax.experimental.pallas import tpu_sc as plsc

N = 10000
E = 320000
D = 128
NC = 2
NS = 16
NW = NC * NS
L = 16
C = 128
EPT = E // NW
EPT_PAD = 10240
NPADE = EPT_PAD - EPT
NCHUNK = EPT_PAD // C
NPAD = 10240
RPT = NPAD // NS
HALF = 5120
DUMP = 512
ACCR = HALF + DUMP
ZR = ACCR // NS // 4
WPT = HALF // NS

_MESH = dict(core_axis_name="c", subcore_axis_name="s")


def _sc_compiler_params():
    cp = pltpu.CompilerParams()
    if "needs_layout_passes" in pltpu.CompilerParams.__dataclass_fields__:
        cp = dataclasses.replace(cp, needs_layout_passes=False)
    return cp


def _proj_tc(x_pad, W, a_l, a_r):
    def body(x_ref, w_ref, al_ref, ar_ref, feat_ref, el_ref, er_ref):
        feat = jnp.dot(x_ref[...], w_ref[...], preferred_element_type=jnp.float32)
        feat_ref[...] = feat
        el_ref[...] = jnp.dot(feat, al_ref[...], preferred_element_type=jnp.float32)
        er_ref[...] = jnp.dot(feat, ar_ref[...], preferred_element_type=jnp.float32)

    feat, el, er = pl.pallas_call(
        body,
        out_shape=(
            jax.ShapeDtypeStruct((NPAD, D), jnp.float32),
            jax.ShapeDtypeStruct((NPAD, 1), jnp.float32),
            jax.ShapeDtypeStruct((NPAD, 1), jnp.float32),
        ),
    )(x_pad, W, a_l.reshape(D, 1), a_r.reshape(D, 1))
    return feat, el[:, 0], er[:, 0]


def _combine_elu_tc(p):
    def body(p_ref, h_ref):
        for hh in range(2):
            z = p_ref[0, hh] + p_ref[1, hh]
            h_ref[hh * HALF:(hh + 1) * HALF, :] = jnp.where(
                z > 0, z, jnp.exp(jnp.minimum(z, 0.0)) - 1.0)

    return pl.pallas_call(
        body,
        out_shape=jax.ShapeDtypeStruct((NPAD, D), jnp.float32),
    )(p)


def _combine_final_tc(p):
    def body(p_ref, o_ref):
        o_ref[:HALF, :] = p_ref[0, 0] + p_ref[1, 0]
        o_ref[HALF:, :] = (p_ref[0, 1, :N - HALF, :] + p_ref[1, 1, :N - HALF, :])

    return pl.pallas_call(
        body,
        out_shape=jax.ShapeDtypeStruct((N, D), jnp.float32),
    )(p)


def _sc_scores(el, er, src_r, dst_r):
    mesh = plsc.VectorSubcoreMesh(**_MESH)

    @functools.partial(
        pl.kernel,
        out_type=(
            jax.ShapeDtypeStruct((NW, NCHUNK, C), jnp.float32),
            jax.ShapeDtypeStruct((NC, NPAD), jnp.float32),
        ),
        mesh=mesh,
        compiler_params=_sc_compiler_params(),
        scratch_types=[
            pltpu.VMEM((NPAD,), jnp.float32),
            pltpu.VMEM((NPAD,), jnp.float32),
            pltpu.VMEM((NCHUNK, C), jnp.int32),
            pltpu.VMEM((NCHUNK, C), jnp.int32),
            pltpu.VMEM((NCHUNK, C), jnp.float32),
            pltpu.VMEM((RPT,), jnp.float32),
            pltpu.VMEM_SHARED((NPAD,), jnp.float32),
        ],
    )
    def k(el_hbm, er_hbm, src_hbm, dst_hbm, w_hbm, s_hbm,
          el_v, er_v, src_v, dst_v, w_v, z_v, s_sh):
        cid = lax.axis_index("c")
        sid = lax.axis_index("s")
        wid = cid * NS + sid

        zero16 = jnp.zeros((L,), jnp.float32)

        @pl.loop(0, RPT // L)
        def _(i):
            z_v[pl.ds(i * L, L)] = zero16

        pltpu.sync_copy(z_v, s_sh.at[pl.ds(sid * RPT, RPT)])

        pltpu.sync_copy(el_hbm, el_v)
        pltpu.sync_copy(er_hbm, er_v)
        pltpu.sync_copy(src_hbm.at[wid], src_v)
        pltpu.sync_copy(dst_hbm.at[wid], dst_v)

        plsc.subcore_barrier()

        @pl.loop(0, NCHUNK)
        def _(j):
            for kk in range(C // L):
                sl = pl.ds(kk * L, L)
                s16 = src_v[j, sl]
                d16 = dst_v[j, sl]
                z = plsc.load_gather(el_v, [s16]) + plsc.load_gather(er_v, [d16])
                e = jnp.where(z >= 0, z, 0.2 * z)
                w_v[j, sl] = jnp.exp(e)
            pltpu.sync_copy(w_v.at[j], s_sh.at[dst_v.at[j]], add=True)

        pltpu.sync_copy(w_v, w_hbm.at[wid])

        plsc.subcore_barrier()

        pltpu.sync_copy(s_sh.at[pl.ds(sid * RPT, RPT)],
                        s_hbm.at[cid, pl.ds(sid * RPT, RPT)])

    return k(el, er, src_r, dst_r)


def _sc_aggregate(featp, w, s, src_r, dst_r):
    mesh = plsc.VectorSubcoreMesh(**_MESH)

    @functools.partial(
        pl.kernel,
        out_type=jax.ShapeDtypeStruct((NC, 2, HALF, D), jnp.float32),
        mesh=mesh,
        compiler_params=_sc_compiler_params(),
        scratch_types=[
            pltpu.VMEM((NPAD,), jnp.float32),
            pltpu.VMEM((NPAD,), jnp.float32),
            pltpu.VMEM((NCHUNK, C), jnp.int32),
            pltpu.VMEM((NCHUNK, C), jnp.int32),
            pltpu.VMEM((NCHUNK, C), jnp.float32),
            pltpu.VMEM((C,), jnp.int32),
            pltpu.VMEM((C,), jnp.int32),
            pltpu.VMEM((C, D), jnp.float32),
            pltpu.VMEM((C, D), jnp.float32),
            pltpu.SemaphoreType.DMA,
            pltpu.SemaphoreType.DMA,
            pltpu.SemaphoreType.DMA,
            pltpu.SemaphoreType.DMA,
            pltpu.VMEM_SHARED((ACCR, D), jnp.float32),
        ],
    )
    def k(feat_hbm, w_hbm, s_hbm, src_hbm, dst_hbm, out_hbm,
          sinv_v, tmp_v, src_v, dst_v, al_v,
          idx0, idx1, rows0, rows1,
          semg0, semg1, sems0, sems1, acc_sh):
        idx = (idx0, idx1)
        rows = (rows0, rows1)
        semg = (semg0, semg1)
        sems = (sems0, sems1)
        cid = lax.axis_index("c")
        sid = lax.axis_index("s")
        wid = cid * NS + sid

        zero16 = jnp.zeros((L,), jnp.float32)

        def zero_rows0():
            @pl.loop(0, C)
            def _(r):
                for cc in range(D // L):
                    rows[0][r, pl.ds(cc * L, L)] = zero16

        def zero_acc():
            base = sid * (ACCR // NS)
            for off, nrow in ((0, C), (C, C), (2 * C, ACCR // NS - 2 * C)):
                pltpu.sync_copy(rows[0].at[pl.ds(0, nrow)],
                                acc_sh.at[pl.ds(base + off, nrow)])

        zero_rows0()
        zero_acc()

        pltpu.sync_copy(s_hbm.at[0], sinv_v)
        pltpu.sync_copy(s_hbm.at[1], tmp_v)

        @pl.loop(0, NPAD // L)
        def _(i):
            sl = pl.ds(i * L, L)
            sinv_v[sl] = 1.0 / (sinv_v[sl] + tmp_v[sl] + 1e-9)

        pltpu.sync_copy(src_hbm.at[wid], src_v)
        pltpu.sync_copy(dst_hbm.at[wid], dst_v)
        pltpu.sync_copy(w_hbm.at[wid], al_v)

        @pl.loop(0, NCHUNK)
        def _(j):
            for kk in range(C // L):
                sl = pl.ds(kk * L, L)
                g = plsc.load_gather(sinv_v, [dst_v[j, sl]])
                al_v[j, sl] = al_v[j, sl] * g

        plsc.subcore_barrier()

        iota16 = lax.iota(jnp.int32, L)

        def gather_issue(j, p):
            pltpu.async_copy(feat_hbm.at[src_v.at[j]], rows[p], semg[p])

        def gather_wait(j, p):
            pltpu.make_async_copy(feat_hbm.at[src_v.at[j]], rows[p],
                                  semg[p]).wait()

        def scatter_issue(j, p):
            pltpu.async_copy(rows[p], acc_sh.at[idx[p]], sems[p], add=True)

        def scatter_drain(p):
            pltpu.make_async_copy(rows[p], acc_sh.at[idx[p]], sems[p]).wait()

        for h in range(2):
            gather_issue(0, 0)

            @pl.loop(0, NCHUNK // 2)
            def _(t):
                base = t * 2
                for p in range(2):
                    j = base + p
                    gather_wait(j, p)

                    pn = (p + 1) % 2

                    @pl.when(j + 1 < NCHUNK)
                    def _():
                        gather_issue(j + 1, pn)

                    for kk in range(C // L):
                        sl = pl.ds(kk * L, L)
                        local = dst_v[j, sl] - (h * HALF)
                        dump = HALF + ((j * C + kk * L + iota16) & (DUMP - 1))
                        ok = (local >= 0) & (local < HALF)
                        idx[p][sl] = jnp.where(ok, local, dump)

                    @plsc.parallel_loop(0, C, unroll=4)
                    def _(r):
                        a16 = plsc.load_gather(al_v.at[j],
                                               [jnp.full((L,), r, jnp.int32)])
                        for cc in range(D // L):
                            sl = pl.ds(cc * L, L)
                            rows[p][r, sl] = rows[p][r, sl] * a16

                    pltpu.sync_copy(rows[p], acc_sh.at[idx[p]], add=True)

            plsc.subcore_barrier()

            pltpu.sync_copy(acc_sh.at[pl.ds(sid * WPT, WPT)],
                            out_hbm.at[cid, h, pl.ds(sid * WPT, WPT)])

            if h == 0:
                zero_rows0()
                zero_acc()
                plsc.subcore_barrier()

    return k(featp, w, s, src_r, dst_r)


def _gat_layer(x_pad, src_r, dst_r, W, a_l, a_r):
    featp, el, er = _proj_tc(x_pad, W, a_l, a_r)
    w, s = _sc_scores(el, er, src_r, dst_r)
    return _sc_aggregate(featp, w, s, src_r, dst_r)


def kernel(x, edge_index, W0, a_l0, a_r0, W1, a_l1, a_r1):
    x_pad = jnp.pad(x, ((0, NPAD - N), (0, 0)))
    src_r = jnp.concatenate(
        [edge_index[0].reshape(NW, EPT),
         jnp.zeros((NW, NPADE), jnp.int32)], axis=1).reshape(NW, NCHUNK, C)
    dst_r = jnp.concatenate(
        [edge_index[1].reshape(NW, EPT),
         jnp.full((NW, NPADE), N, jnp.int32)], axis=1).reshape(NW, NCHUNK, C)
    p0 = _gat_layer(x_pad, src_r, dst_r, W0, a_l0, a_r0)
    h = _combine_elu_tc(p0)
    p1 = _gat_layer(h, src_r, dst_r, W1, a_l1, a_r1)
    return _combine_final_tc(p1)


# speedup vs baseline: 1.0909x; 1.0000x over previous
import dataclasses
import functools

import jax
import jax.numpy as jnp
from jax import lax
from jax.experimental import pallas as pl
from jax.experimental.pallas import tpu as pltpu
from jax.experimental.pallas import tpu_sc as plsc

N = 10000
E = 320000
D = 128
NC = 2
NS = 16
NW = NC * NS
L = 16
C = 128
EPT = E // NW
EPT_PAD = 10240
NPADE = EPT_PAD - EPT
NCHUNK = EPT_PAD // C
NPAD = 10240
RPT = NPAD // NS
HALF = 5120
DUMP = 512
ACCR = HALF + DUMP
ZR = ACCR // NS // 4
WPT = HALF // NS

_MESH = dict(core_axis_name="c", subcore_axis_name="s")


def _sc_compiler_params():
    cp = pltpu.CompilerParams()
    if "needs_layout_passes" in pltpu.CompilerParams.__dataclass_fields__:
        cp = dataclasses.replace(cp, needs_layout_passes=False)
    return cp


def _proj_tc(x_pad, W, a_l, a_r):
    def body(x_ref, w_ref, al_ref, ar_ref, feat_ref, el_ref, er_ref):
        feat = jnp.dot(x_ref[...], w_ref[...], preferred_element_type=jnp.float32)
        feat_ref[...] = feat
        el_ref[...] = jnp.dot(feat, al_ref[...], preferred_element_type=jnp.float32)
        er_ref[...] = jnp.dot(feat, ar_ref[...], preferred_element_type=jnp.float32)

    feat, el, er = pl.pallas_call(
        body,
        out_shape=(
            jax.ShapeDtypeStruct((NPAD, D), jnp.float32),
            jax.ShapeDtypeStruct((NPAD, 1), jnp.float32),
            jax.ShapeDtypeStruct((NPAD, 1), jnp.float32),
        ),
    )(x_pad, W, a_l.reshape(D, 1), a_r.reshape(D, 1))
    return feat, el[:, 0], er[:, 0]


def _combine_elu_tc(p):
    def body(p_ref, h_ref):
        for hh in range(2):
            z = p_ref[0, hh] + p_ref[1, hh]
            h_ref[hh * HALF:(hh + 1) * HALF, :] = jnp.where(
                z > 0, z, jnp.exp(jnp.minimum(z, 0.0)) - 1.0)

    return pl.pallas_call(
        body,
        out_shape=jax.ShapeDtypeStruct((NPAD, D), jnp.float32),
    )(p)


def _combine_final_tc(p):
    def body(p_ref, o_ref):
        o_ref[:HALF, :] = p_ref[0, 0] + p_ref[1, 0]
        o_ref[HALF:, :] = (p_ref[0, 1, :N - HALF, :] + p_ref[1, 1, :N - HALF, :])

    return pl.pallas_call(
        body,
        out_shape=jax.ShapeDtypeStruct((N, D), jnp.float32),
    )(p)


def _sc_scores(el, er, src_r, dst_r):
    mesh = plsc.VectorSubcoreMesh(**_MESH)

    @functools.partial(
        pl.kernel,
        out_type=(
            jax.ShapeDtypeStruct((NW, NCHUNK, C), jnp.float32),
            jax.ShapeDtypeStruct((NC, NPAD), jnp.float32),
        ),
        mesh=mesh,
        compiler_params=_sc_compiler_params(),
        scratch_types=[
            pltpu.VMEM((NPAD,), jnp.float32),
            pltpu.VMEM((NPAD,), jnp.float32),
            pltpu.VMEM((NCHUNK, C), jnp.int32),
            pltpu.VMEM((NCHUNK, C), jnp.int32),
            pltpu.VMEM((NCHUNK, C), jnp.float32),
            pltpu.VMEM((RPT,), jnp.float32),
            pltpu.VMEM_SHARED((NPAD,), jnp.float32),
        ],
    )
    def k(el_hbm, er_hbm, src_hbm, dst_hbm, w_hbm, s_hbm,
          el_v, er_v, src_v, dst_v, w_v, z_v, s_sh):
        cid = lax.axis_index("c")
        sid = lax.axis_index("s")
        wid = cid * NS + sid

        zero16 = jnp.zeros((L,), jnp.float32)

        @pl.loop(0, RPT // L)
        def _(i):
            z_v[pl.ds(i * L, L)] = zero16

        pltpu.sync_copy(z_v, s_sh.at[pl.ds(sid * RPT, RPT)])

        pltpu.sync_copy(el_hbm, el_v)
        pltpu.sync_copy(er_hbm, er_v)
        pltpu.sync_copy(src_hbm.at[wid], src_v)
        pltpu.sync_copy(dst_hbm.at[wid], dst_v)

        plsc.subcore_barrier()

        @pl.loop(0, NCHUNK)
        def _(j):
            for kk in range(C // L):
                sl = pl.ds(kk * L, L)
                s16 = src_v[j, sl]
                d16 = dst_v[j, sl]
                z = plsc.load_gather(el_v, [s16]) + plsc.load_gather(er_v, [d16])
                e = jnp.where(z >= 0, z, 0.2 * z)
                w_v[j, sl] = jnp.exp(e)
            pltpu.sync_copy(w_v.at[j], s_sh.at[dst_v.at[j]], add=True)

        pltpu.sync_copy(w_v, w_hbm.at[wid])

        plsc.subcore_barrier()

        pltpu.sync_copy(s_sh.at[pl.ds(sid * RPT, RPT)],
                        s_hbm.at[cid, pl.ds(sid * RPT, RPT)])

    return k(el, er, src_r, dst_r)


def _sc_aggregate(featp, w, s, src_r, dst_r):
    mesh = plsc.VectorSubcoreMesh(**_MESH)

    @functools.partial(
        pl.kernel,
        out_type=jax.ShapeDtypeStruct((NC, 2, HALF, D), jnp.float32),
        mesh=mesh,
        compiler_params=_sc_compiler_params(),
        scratch_types=[
            pltpu.VMEM((NPAD,), jnp.float32),
            pltpu.VMEM((NPAD,), jnp.float32),
            pltpu.VMEM((NCHUNK, C), jnp.int32),
            pltpu.VMEM((NCHUNK, C), jnp.int32),
            pltpu.VMEM((NCHUNK, C), jnp.float32),
            pltpu.VMEM((C,), jnp.int32),
            pltpu.VMEM((C,), jnp.int32),
            pltpu.VMEM((C, D), jnp.float32),
            pltpu.VMEM((C, D), jnp.float32),
            pltpu.SemaphoreType.DMA,
            pltpu.SemaphoreType.DMA,
            pltpu.SemaphoreType.DMA,
            pltpu.SemaphoreType.DMA,
            pltpu.VMEM_SHARED((ACCR, D), jnp.float32),
        ],
    )
    def k(feat_hbm, w_hbm, s_hbm, src_hbm, dst_hbm, out_hbm,
          sinv_v, tmp_v, src_v, dst_v, al_v,
          idx0, idx1, rows0, rows1,
          semg0, semg1, sems0, sems1, acc_sh):
        idx = (idx0, idx1)
        rows = (rows0, rows1)
        semg = (semg0, semg1)
        sems = (sems0, sems1)
        cid = lax.axis_index("c")
        sid = lax.axis_index("s")
        wid = cid * NS + sid

        zero16 = jnp.zeros((L,), jnp.float32)

        def zero_rows0():
            @pl.loop(0, C)
            def _(r):
                for cc in range(D // L):
                    rows[0][r, pl.ds(cc * L, L)] = zero16

        def zero_acc():
            base = sid * (ACCR // NS)
            for off, nrow in ((0, C), (C, C), (2 * C, ACCR // NS - 2 * C)):
                pltpu.sync_copy(rows[0].at[pl.ds(0, nrow)],
                                acc_sh.at[pl.ds(base + off, nrow)])

        zero_rows0()
        zero_acc()

        pltpu.sync_copy(s_hbm.at[0], sinv_v)
        pltpu.sync_copy(s_hbm.at[1], tmp_v)

        @pl.loop(0, NPAD // L)
        def _(i):
            sl = pl.ds(i * L, L)
            sinv_v[sl] = 1.0 / (sinv_v[sl] + tmp_v[sl] + 1e-9)

        pltpu.sync_copy(src_hbm.at[wid], src_v)
        pltpu.sync_copy(dst_hbm.at[wid], dst_v)
        pltpu.sync_copy(w_hbm.at[wid], al_v)

        @pl.loop(0, NCHUNK)
        def _(j):
            for kk in range(C // L):
                sl = pl.ds(kk * L, L)
                g = plsc.load_gather(sinv_v, [dst_v[j, sl]])
                al_v[j, sl] = al_v[j, sl] * g

        plsc.subcore_barrier()

        iota16 = lax.iota(jnp.int32, L)

        QS = C // 4

        def gather_issue(j, p):
            for q in range(4):
                sl = pl.ds(q * QS, QS)
                pltpu.async_copy(feat_hbm.at[src_v.at[j, sl]],
                                 rows[p].at[sl], semg[p])

        def gather_wait(j, p):
            for q in range(4):
                sl = pl.ds(q * QS, QS)
                pltpu.make_async_copy(feat_hbm.at[src_v.at[j, sl]],
                                      rows[p].at[sl], semg[p]).wait()

        def scatter_issue(j, p):
            pltpu.async_copy(rows[p], acc_sh.at[idx[p]], sems[p], add=True)

        def scatter_drain(p):
            pltpu.make_async_copy(rows[p], acc_sh.at[idx[p]], sems[p]).wait()

        for h in range(2):
            gather_issue(0, 0)

            @pl.loop(0, NCHUNK // 2)
            def _(t):
                base = t * 2
                for p in range(2):
                    j = base + p
                    gather_wait(j, p)

                    pn = (p + 1) % 2

                    @pl.when(j + 1 < NCHUNK)
                    def _():
                        gather_issue(j + 1, pn)

                    for kk in range(C // L):
                        sl = pl.ds(kk * L, L)
                        local = dst_v[j, sl] - (h * HALF)
                        dump = HALF + ((j * C + kk * L + iota16) & (DUMP - 1))
                        ok = (local >= 0) & (local < HALF)
                        idx[p][sl] = jnp.where(ok, local, dump)

                    @plsc.parallel_loop(0, C, unroll=4)
                    def _(r):
                        a16 = plsc.load_gather(al_v.at[j],
                                               [jnp.full((L,), r, jnp.int32)])
                        for cc in range(D // L):
                            sl = pl.ds(cc * L, L)
                            rows[p][r, sl] = rows[p][r, sl] * a16

                    pltpu.sync_copy(rows[p], acc_sh.at[idx[p]], add=True)

            plsc.subcore_barrier()

            pltpu.sync_copy(acc_sh.at[pl.ds(sid * WPT, WPT)],
                            out_hbm.at[cid, h, pl.ds(sid * WPT, WPT)])

            if h == 0:
                zero_rows0()
                zero_acc()
                plsc.subcore_barrier()

    return k(featp, w, s, src_r, dst_r)


def _gat_layer(x_pad, src_r, dst_r, W, a_l, a_r):
    featp, el, er = _proj_tc(x_pad, W, a_l, a_r)
    w, s = _sc_scores(el, er, src_r, dst_r)
    return _sc_aggregate(featp, w, s, src_r, dst_r)


def kernel(x, edge_index, W0, a_l0, a_r0, W1, a_l1, a_r1):
    x_pad = jnp.pad(x, ((0, NPAD - N), (0, 0)))
    src_r = jnp.concatenate(
        [edge_index[0].reshape(NW, EPT),
         jnp.zeros((NW, NPADE), jnp.int32)], axis=1).reshape(NW, NCHUNK, C)
    dst_r = jnp.concatenate(
        [edge_index[1].reshape(NW, EPT),
         jnp.full((NW, NPADE), N, jnp.int32)], axis=1).reshape(NW, NCHUNK, C)
    p0 = _gat_layer(x_pad, src_r, dst_r, W0, a_l0, a_r0)
    h = _combine_elu_tc(p0)
    p1 = _gat_layer(h, src_r, dst_r, W1, a_l1, a_r1)
    return _combine_final_tc(p1)


# speedup vs baseline: 1.0918x; 1.0009x over previous
import dataclasses
import functools

import jax
import jax.numpy as jnp
from jax import lax
from jax.experimental import pallas as pl
from jax.experimental.pallas import tpu as pltpu
from jax.experimental.pallas import tpu_sc as plsc

N = 10000
E = 320000
D = 128
NC = 2
NS = 16
NW = NC * NS
L = 16
C = 128
EPT = E // NW
EPT_PAD = 10240
NPADE = EPT_PAD - EPT
NCHUNK = EPT_PAD // C
NPAD = 10240
RPT = NPAD // NS
HALF = 5120
DUMP = 512
ACCR = HALF + DUMP
ZR = ACCR // NS // 4
WPT = HALF // NS

_MESH = dict(core_axis_name="c", subcore_axis_name="s")


def _sc_compiler_params():
    cp = pltpu.CompilerParams()
    if "needs_layout_passes" in pltpu.CompilerParams.__dataclass_fields__:
        cp = dataclasses.replace(cp, needs_layout_passes=False)
    return cp


def _proj_tc(x_pad, W, a_l, a_r):
    def body(x_ref, w_ref, al_ref, ar_ref, feat_ref, el_ref, er_ref):
        feat = jnp.dot(x_ref[...], w_ref[...], preferred_element_type=jnp.float32)
        feat_ref[...] = feat
        el_ref[...] = jnp.dot(feat, al_ref[...], preferred_element_type=jnp.float32)
        er_ref[...] = jnp.dot(feat, ar_ref[...], preferred_element_type=jnp.float32)

    feat, el, er = pl.pallas_call(
        body,
        out_shape=(
            jax.ShapeDtypeStruct((NPAD, D), jnp.float32),
            jax.ShapeDtypeStruct((NPAD, 1), jnp.float32),
            jax.ShapeDtypeStruct((NPAD, 1), jnp.float32),
        ),
    )(x_pad, W, a_l.reshape(D, 1), a_r.reshape(D, 1))
    return feat, el[:, 0], er[:, 0]


def _combine_elu_tc(p):
    def body(p_ref, h_ref):
        for hh in range(2):
            z = p_ref[0, hh] + p_ref[1, hh]
            h_ref[hh * HALF:(hh + 1) * HALF, :] = jnp.where(
                z > 0, z, jnp.exp(jnp.minimum(z, 0.0)) - 1.0)

    return pl.pallas_call(
        body,
        out_shape=jax.ShapeDtypeStruct((NPAD, D), jnp.float32),
    )(p)


def _combine_final_tc(p):
    def body(p_ref, o_ref):
        o_ref[:HALF, :] = p_ref[0, 0] + p_ref[1, 0]
        o_ref[HALF:, :] = (p_ref[0, 1, :N - HALF, :] + p_ref[1, 1, :N - HALF, :])

    return pl.pallas_call(
        body,
        out_shape=jax.ShapeDtypeStruct((N, D), jnp.float32),
    )(p)


def _sc_scores(el, er, src_r, dst_r):
    mesh = plsc.VectorSubcoreMesh(**_MESH)

    @functools.partial(
        pl.kernel,
        out_type=(
            jax.ShapeDtypeStruct((NW, NCHUNK, C), jnp.float32),
            jax.ShapeDtypeStruct((NC, NPAD), jnp.float32),
        ),
        mesh=mesh,
        compiler_params=_sc_compiler_params(),
        scratch_types=[
            pltpu.VMEM((NPAD,), jnp.float32),
            pltpu.VMEM((NPAD,), jnp.float32),
            pltpu.VMEM((NCHUNK, C), jnp.int32),
            pltpu.VMEM((NCHUNK, C), jnp.int32),
            pltpu.VMEM((NCHUNK, C), jnp.float32),
            pltpu.VMEM((RPT,), jnp.float32),
            pltpu.VMEM_SHARED((NPAD,), jnp.float32),
        ],
    )
    def k(el_hbm, er_hbm, src_hbm, dst_hbm, w_hbm, s_hbm,
          el_v, er_v, src_v, dst_v, w_v, z_v, s_sh):
        cid = lax.axis_index("c")
        sid = lax.axis_index("s")
        wid = cid * NS + sid

        zero16 = jnp.zeros((L,), jnp.float32)

        @pl.loop(0, RPT // L)
        def _(i):
            z_v[pl.ds(i * L, L)] = zero16

        pltpu.sync_copy(z_v, s_sh.at[pl.ds(sid * RPT, RPT)])

        pltpu.sync_copy(el_hbm, el_v)
        pltpu.sync_copy(er_hbm, er_v)
        pltpu.sync_copy(src_hbm.at[wid], src_v)
        pltpu.sync_copy(dst_hbm.at[wid], dst_v)

        plsc.subcore_barrier()

        @pl.loop(0, NCHUNK)
        def _(j):
            for kk in range(C // L):
                sl = pl.ds(kk * L, L)
                s16 = src_v[j, sl]
                d16 = dst_v[j, sl]
                z = plsc.load_gather(el_v, [s16]) + plsc.load_gather(er_v, [d16])
                e = jnp.where(z >= 0, z, 0.2 * z)
                w_v[j, sl] = jnp.exp(e)
            pltpu.sync_copy(w_v.at[j], s_sh.at[dst_v.at[j]], add=True)

        pltpu.sync_copy(w_v, w_hbm.at[wid])

        plsc.subcore_barrier()

        pltpu.sync_copy(s_sh.at[pl.ds(sid * RPT, RPT)],
                        s_hbm.at[cid, pl.ds(sid * RPT, RPT)])

    return k(el, er, src_r, dst_r)


def _sc_aggregate(featp, w, s, src_r, dst_r):
    mesh = plsc.VectorSubcoreMesh(**_MESH)

    @functools.partial(
        pl.kernel,
        out_type=jax.ShapeDtypeStruct((NC, 2, HALF, D), jnp.float32),
        mesh=mesh,
        compiler_params=_sc_compiler_params(),
        scratch_types=[
            pltpu.VMEM((NPAD,), jnp.float32),
            pltpu.VMEM((NPAD,), jnp.float32),
            pltpu.VMEM((NCHUNK, C), jnp.int32),
            pltpu.VMEM((NCHUNK, C), jnp.int32),
            pltpu.VMEM((NCHUNK, C), jnp.float32),
            pltpu.VMEM((C,), jnp.int32),
            pltpu.VMEM((C,), jnp.int32),
            pltpu.VMEM((C, D), jnp.float32),
            pltpu.VMEM((C, D), jnp.float32),
            pltpu.SemaphoreType.DMA,
            pltpu.SemaphoreType.DMA,
            pltpu.SemaphoreType.DMA,
            pltpu.SemaphoreType.DMA,
            pltpu.VMEM_SHARED((ACCR, D), jnp.float32),
        ],
    )
    def k(feat_hbm, w_hbm, s_hbm, src_hbm, dst_hbm, out_hbm,
          sinv_v, tmp_v, src_v, dst_v, al_v,
          idx0, idx1, rows0, rows1,
          semg0, semg1, sems0, sems1, acc_sh):
        idx = (idx0, idx1)
        rows = (rows0, rows1)
        semg = (semg0, semg1)
        sems = (sems0, sems1)
        cid = lax.axis_index("c")
        sid = lax.axis_index("s")
        wid = cid * NS + sid

        zero16 = jnp.zeros((L,), jnp.float32)

        def zero_rows0():
            @pl.loop(0, C)
            def _(r):
                for cc in range(D // L):
                    rows[0][r, pl.ds(cc * L, L)] = zero16

        def zero_acc():
            base = sid * (ACCR // NS)
            for off, nrow in ((0, C), (C, C), (2 * C, ACCR // NS - 2 * C)):
                pltpu.sync_copy(rows[0].at[pl.ds(0, nrow)],
                                acc_sh.at[pl.ds(base + off, nrow)])

        zero_rows0()
        zero_acc()

        pltpu.sync_copy(s_hbm.at[0], sinv_v)
        pltpu.sync_copy(s_hbm.at[1], tmp_v)

        @pl.loop(0, NPAD // L)
        def _(i):
            sl = pl.ds(i * L, L)
            sinv_v[sl] = 1.0 / (sinv_v[sl] + tmp_v[sl] + 1e-9)

        pltpu.sync_copy(src_hbm.at[wid], src_v)
        pltpu.sync_copy(dst_hbm.at[wid], dst_v)
        pltpu.sync_copy(w_hbm.at[wid], al_v)

        @pl.loop(0, NCHUNK)
        def _(j):
            for kk in range(C // L):
                sl = pl.ds(kk * L, L)
                g = plsc.load_gather(sinv_v, [dst_v[j, sl]])
                al_v[j, sl] = al_v[j, sl] * g

        plsc.subcore_barrier()

        iota16 = lax.iota(jnp.int32, L)

        def gather_issue(j, p):
            pltpu.async_copy(feat_hbm.at[src_v.at[j]], rows[p], semg[p])

        def gather_wait(j, p):
            pltpu.make_async_copy(feat_hbm.at[src_v.at[j]], rows[p],
                                  semg[p]).wait()

        def scatter_issue(j, p):
            pltpu.async_copy(rows[p], acc_sh.at[idx[p]], sems[p], add=True)

        def scatter_drain(p):
            pltpu.make_async_copy(rows[p], acc_sh.at[idx[p]], sems[p]).wait()

        for h in range(2):
            gather_issue(0, 0)

            @pl.loop(0, NCHUNK // 2)
            def _(t):
                base = t * 2
                for p in range(2):
                    j = base + p
                    gather_wait(j, p)

                    pn = (p + 1) % 2

                    @pl.when(j + 1 < NCHUNK)
                    def _():
                        gather_issue(j + 1, pn)

                    for kk in range(C // L):
                        sl = pl.ds(kk * L, L)
                        local = dst_v[j, sl] - (h * HALF)
                        dump = HALF + ((j * C + kk * L + iota16) & (DUMP - 1))
                        ok = (local >= 0) & (local < HALF)
                        idx[p][sl] = jnp.where(ok, local, dump)

                    @plsc.parallel_loop(0, C, unroll=4)
                    def _(r):
                        a16 = plsc.load_gather(al_v.at[j],
                                               [jnp.full((L,), r, jnp.int32)])
                        for cc in range(D // L):
                            sl = pl.ds(cc * L, L)
                            rows[p][r, sl] = rows[p][r, sl] * a16

                    pltpu.sync_copy(rows[p], acc_sh.at[idx[p]], add=True)

            plsc.subcore_barrier()

            pltpu.sync_copy(acc_sh.at[pl.ds(sid * WPT, WPT)],
                            out_hbm.at[cid, h, pl.ds(sid * WPT, WPT)])

            if h == 0:
                zero_rows0()
                zero_acc()
                plsc.subcore_barrier()

    return k(featp, w, s, src_r, dst_r)


def _gat_layer(x_pad, src_r, dst_r, W, a_l, a_r):
    featp, el, er = _proj_tc(x_pad, W, a_l, a_r)
    w, s = _sc_scores(el, er, src_r, dst_r)
    return _sc_aggregate(featp, w, s, src_r, dst_r)


def kernel(x, edge_index, W0, a_l0, a_r0, W1, a_l1, a_r1):
    x_pad = jnp.pad(x, ((0, NPAD - N), (0, 0)))
    src_r = jnp.concatenate(
        [edge_index[0].reshape(NW, EPT),
         jnp.zeros((NW, NPADE), jnp.int32)], axis=1).reshape(NW, NCHUNK, C)
    dst_r = jnp.concatenate(
        [edge_index[1].reshape(NW, EPT),
         jnp.full((NW, NPADE), N, jnp.int32)], axis=1).reshape(NW, NCHUNK, C)
    p0 = _gat_layer(x_pad, src_r, dst_r, W0, a_l0, a_r0)
    h = _combine_elu_tc(p0)
    p1 = _gat_layer(h, src_r, dst_r, W1, a_l1, a_r1)
    return _combine_final_tc(p1)
